# Initial kernel scaffold; baseline (speedup 1.0000x reference)
import jax
import jax.numpy as jnp
from jax.experimental import pallas as pl


def kernel(x, edge_index, W1, a_src1, a_dst1, b1, W2, a_src2, a_dst2, b2):
    raise NotImplementedError("write your pallas kernel here")


# speedup vs baseline: 15.4170x; 15.4170x over previous
import functools

import jax
import jax.numpy as jnp
from jax import lax
from jax.experimental import pallas as pl
from jax.experimental.pallas import tpu as pltpu
from jax.experimental.pallas import tpu_sc as plsc

N_NODES = 10000
N_IN, N_HID, N_OUT = 128, 64, 128
N_EDGES = 330000
NW = 32
L = 16
CPT = 10368
NCH = CPT // L
EE_PAD = CPT * NW
N_PAD = 10240
RPT = N_PAD // L
ZR = 16
SEG = 72
NSEG = NCH // SEG


def _edge_pass(D):
    mesh = plsc.VectorSubcoreMesh(core_axis_name="c", subcore_axis_name="s")

    @functools.partial(
        pl.kernel,
        out_type=[
            jax.ShapeDtypeStruct((2 * N_PAD, D), jnp.float32),
            jax.ShapeDtypeStruct((NW * N_NODES,), jnp.float32),
        ],
        mesh=mesh,
        compiler_params=pltpu.CompilerParams(
            needs_layout_passes=False, use_tc_tiling_on_sc=False),
        scratch_types=[
            pltpu.VMEM((N_NODES,), jnp.float32),
            pltpu.VMEM((N_NODES,), jnp.float32),
            pltpu.VMEM((SEG, L), jnp.int32),
            pltpu.VMEM((SEG, L), jnp.int32),
            pltpu.VMEM((SEG * L,), jnp.float32),
            pltpu.VMEM((N_NODES,), jnp.float32),
            pltpu.VMEM((L, D), jnp.float32),
            pltpu.VMEM((ZR, D), jnp.float32),
            pltpu.VMEM_SHARED((N_PAD, D), jnp.float32),
            pltpu.SemaphoreType.DMA,
        ],
    )
    def k(src_hbm, dst_hbm, asv_hbm, adv_hbm, h_hbm, out_part, denom_part,
          as_v, ad_v, src2, dst2, ex_v, denom_v, rows_v, zero_v, out_s, sem):
        cid = lax.axis_index("c")
        sid = lax.axis_index("s")
        wid = sid * 2 + cid

        pltpu.sync_copy(asv_hbm, as_v)
        pltpu.sync_copy(adv_hbm, ad_v)

        def zrow(j, _):
            for d in range(D // L):
                zero_v[j, pl.ds(d * L, L)] = jnp.zeros((L,), jnp.float32)
            return 0
        lax.fori_loop(0, ZR, zrow, 0)

        def zden(j, _):
            denom_v[pl.ds(j * L, L)] = jnp.zeros((L,), jnp.float32)
            return 0
        lax.fori_loop(0, N_NODES // L, zden, 0)

        for z in range(RPT // ZR):
            pltpu.sync_copy(zero_v, out_s.at[pl.ds(sid * RPT + z * ZR, ZR)])

        plsc.subcore_barrier()

        ebase = wid * CPT

        def seg_body(s, _):
            pltpu.sync_copy(
                src_hbm.at[pl.ds(wid * NCH + s * SEG, SEG)], src2)
            pltpu.sync_copy(
                dst_hbm.at[pl.ds(wid * NCH + s * SEG, SEG)], dst2)

            def sbody(j, _):
                sv = src2[j]
                dv = dst2[j]
                e = (plsc.load_gather(as_v, [sv])
                     + plsc.load_gather(ad_v, [dv]))
                e = jnp.where(e >= 0.0, e, 0.2 * e)
                ex = jnp.exp(e)
                gid = ebase + (s * SEG + j) * L + lax.iota(jnp.int32, 16)
                ex = jnp.where(gid < N_EDGES, ex, 0.0)
                ex_v[pl.ds(j * L, L)] = ex
                plsc.addupdate_scatter(denom_v, [dv], ex)
                return 0
            lax.fori_loop(0, SEG, sbody, 0)

            def vbody(j, _):
                pltpu.async_copy(h_hbm.at[src2.at[j]], rows_v, sem).wait()
                for g in range(L):
                    w = plsc.load_gather(
                        ex_v, [j * L + g + jnp.zeros((L,), jnp.int32)])
                    for d in range(D // L):
                        rows_v[g, pl.ds(d * L, L)] = (
                            rows_v[g, pl.ds(d * L, L)] * w)
                pltpu.sync_copy(rows_v, out_s.at[dst2.at[j]], add=True)
                return 0
            lax.fori_loop(0, SEG, vbody, 0)
            return 0
        lax.fori_loop(0, NSEG, seg_body, 0)

        plsc.subcore_barrier()

        pltpu.sync_copy(out_s.at[pl.ds(sid * RPT, RPT)],
                        out_part.at[pl.ds(cid * N_PAD + sid * RPT, RPT)])
        pltpu.sync_copy(denom_v, denom_part.at[pl.ds(wid * N_NODES, N_NODES)])

    return k


_edge64 = _edge_pass(N_HID)
_edge128 = _edge_pass(N_OUT)

_BLK = 400
_GRID = N_NODES // _BLK


def _head_call(in_d, out_d):
    def body(x_ref, w_ref, a_ref, h_ref, asd_ref):
        h = jnp.dot(x_ref[...], w_ref[...], preferred_element_type=jnp.float32)
        h_ref[...] = h
        asd_ref[...] = jnp.dot(h, a_ref[...],
                               preferred_element_type=jnp.float32)

    return pl.pallas_call(
        body,
        grid=(_GRID,),
        in_specs=[
            pl.BlockSpec((_BLK, in_d), lambda i: (i, 0)),
            pl.BlockSpec((in_d, out_d), lambda i: (0, 0)),
            pl.BlockSpec((out_d, 2), lambda i: (0, 0)),
        ],
        out_specs=[
            pl.BlockSpec((_BLK, out_d), lambda i: (i, 0)),
            pl.BlockSpec((_BLK, 2), lambda i: (i, 0)),
        ],
        out_shape=[
            jax.ShapeDtypeStruct((N_NODES, out_d), jnp.float32),
            jax.ShapeDtypeStruct((N_NODES, 2), jnp.float32),
        ],
    )


_head1 = _head_call(N_IN, N_HID)


def _mid_body(p0_ref, p1_ref, dp_ref, b_ref, w_ref, a_ref, h2_ref, asd_ref):
    den = jnp.sum(dp_ref[...], axis=1)
    h2 = (p0_ref[...] + p1_ref[...]) / (den[:, None] + 1e-16) + b_ref[...]
    h2 = jnp.where(h2 > 0.0, h2, jnp.exp(jnp.minimum(h2, 0.0)) - 1.0)
    hl2 = jnp.dot(h2, w_ref[...], preferred_element_type=jnp.float32)
    h2_ref[...] = hl2
    asd_ref[...] = jnp.dot(hl2, a_ref[...],
                           preferred_element_type=jnp.float32)


_mid = pl.pallas_call(
    _mid_body,
    grid=(_GRID,),
    in_specs=[
        pl.BlockSpec((_BLK, N_HID), lambda i: (i, 0)),
        pl.BlockSpec((_BLK, N_HID), lambda i: (i, 0)),
        pl.BlockSpec((_BLK, NW), lambda i: (i, 0)),
        pl.BlockSpec((1, N_HID), lambda i: (0, 0)),
        pl.BlockSpec((N_HID, N_OUT), lambda i: (0, 0)),
        pl.BlockSpec((N_OUT, 2), lambda i: (0, 0)),
    ],
    out_specs=[
        pl.BlockSpec((_BLK, N_OUT), lambda i: (i, 0)),
        pl.BlockSpec((_BLK, 2), lambda i: (i, 0)),
    ],
    out_shape=[
        jax.ShapeDtypeStruct((N_NODES, N_OUT), jnp.float32),
        jax.ShapeDtypeStruct((N_NODES, 2), jnp.float32),
    ],
)


def _fin_body(p0_ref, p1_ref, dp_ref, b_ref, o_ref):
    den = jnp.sum(dp_ref[...], axis=1)
    o_ref[...] = ((p0_ref[...] + p1_ref[...]) / (den[:, None] + 1e-16)
                  + b_ref[...])


_fin = pl.pallas_call(
    _fin_body,
    grid=(_GRID,),
    in_specs=[
        pl.BlockSpec((_BLK, N_OUT), lambda i: (i, 0)),
        pl.BlockSpec((_BLK, N_OUT), lambda i: (i, 0)),
        pl.BlockSpec((_BLK, NW), lambda i: (i, 0)),
        pl.BlockSpec((1, N_OUT), lambda i: (0, 0)),
    ],
    out_specs=pl.BlockSpec((_BLK, N_OUT), lambda i: (i, 0)),
    out_shape=jax.ShapeDtypeStruct((N_NODES, N_OUT), jnp.float32),
)


def kernel(x, edge_index, W1, a_src1, a_dst1, b1, W2, a_src2, a_dst2, b2):
    loop = jnp.arange(N_NODES, dtype=edge_index.dtype)
    pad = jnp.zeros((EE_PAD - N_EDGES,), edge_index.dtype)
    src = jnp.concatenate([edge_index[0], loop, pad]).reshape(EE_PAD // L, L)
    dst = jnp.concatenate([edge_index[1], loop, pad]).reshape(EE_PAD // L, L)

    A1 = jnp.concatenate(
        [a_src1.reshape(N_HID, 1), a_dst1.reshape(N_HID, 1)], axis=1)
    A2 = jnp.concatenate(
        [a_src2.reshape(N_OUT, 1), a_dst2.reshape(N_OUT, 1)], axis=1)

    h1, asd1 = _head1(x, W1, A1)
    part1, den1 = _edge64(src, dst, asd1[:, 0], asd1[:, 1], h1)
    den1 = den1.reshape(NW, N_NODES).T
    hl2, asd2 = _mid(part1[:N_NODES], part1[N_PAD:N_PAD + N_NODES], den1,
                     b1.reshape(1, N_HID), W2, A2)
    part2, den2 = _edge128(src, dst, asd2[:, 0], asd2[:, 1], hl2)
    den2 = den2.reshape(NW, N_NODES).T
    out = _fin(part2[:N_NODES], part2[N_PAD:N_PAD + N_NODES], den2,
               b2.reshape(1, N_OUT))
    return out
